# baseline (speedup 1.0000x reference)
import functools

import jax
import jax.numpy as jnp
from jax import lax
from jax.experimental import pallas as pl
from jax.experimental.pallas import tpu as pltpu
from jax.experimental.pallas import tpu_sc as plsc

_NUM_GRAPHS = 64
_N_NODES = 50000
_D = 256
_L = 16
_CH = _D // _L

_B = 80
_NC = 2
_NS = 16
_NW = _NC * _NS

_SC_ROWS = 25600
_NBLOCKS = _SC_ROWS // _B
_TC_ROWS = _N_NODES - _SC_ROWS
_TB = 400
_TSTEPS = _TC_ROWS // _TB
_TOFF = _SC_ROWS // _TB


_NBUF = 4


def _sc_body(emb_hbm, batch_hbm,
             sums_hbm, cnts_hbm, stage0_v, stage1_v, stage2_v, stage3_v,
             idx0_v, idx1_v, idx2_v, idx3_v,
             acc_v, cacc_v, sem0, sem1, sem2, sem3):
    cid = lax.axis_index("c")
    sid = lax.axis_index("s")
    wid = sid * _NC + cid

    stages = (stage0_v, stage1_v, stage2_v, stage3_v)
    idxs = (idx0_v, idx1_v, idx2_v, idx3_v)
    sems = (sem0, sem1, sem2, sem3)

    nblk = (_NBLOCKS - 1 - wid) // _NW + 1

    onevec = jnp.ones((_L,), jnp.float32)

    def _copies(i, p):
        base = (wid + i * _NW) * _B
        ci = pltpu.make_async_copy(
            batch_hbm.at[pl.ds(base, _B)], idxs[p], sems[p])
        ce = pltpu.make_async_copy(
            emb_hbm.at[pl.ds(base, _B)], stages[p], sems[p])
        return ci, ce

    zvec = jnp.zeros((_L,), jnp.float32)

    def _accumulate(p, rows):
        stage_v = stages[p]
        idx_v = idxs[p]
        blockvec = jnp.full((_L,), float(rows), jnp.float32)
        chunkvec = jnp.full((_L,), float(_L), jnp.float32)

        g_first = idx_v[pl.ds(0, _L)][0]
        g_last = idx_v[pl.ds(rows - _L, _L)][_L - 1]

        @pl.when(g_first == g_last)
        def _uniform():
            def _row(r, acc):
                return tuple(acc[j] + stage_v[r, pl.ds(j * _L, _L)]
                             for j in range(_CH))
            acc = lax.fori_loop(0, rows, _row, (zvec,) * _CH)
            for j in range(_CH):
                plsc.addupdate(acc_v.at[g_first, pl.ds(j * _L, _L)], acc[j])
            plsc.addupdate(cacc_v.at[g_first, :], blockvec)

        @pl.when(g_first != g_last)
        def _mixed():
            def _chunk16(t, carry):
                gvec = idx_v[pl.ds(t * _L, _L)]
                c_first = gvec[0]
                c_last = gvec[_L - 1]

                @pl.when(c_first == c_last)
                def _chunk_uniform():
                    def _row(r, acc):
                        return tuple(acc[j] + stage_v[r, pl.ds(j * _L, _L)]
                                     for j in range(_CH))
                    acc = lax.fori_loop(t * _L, (t + 1) * _L, _row,
                                        (zvec,) * _CH)
                    for j in range(_CH):
                        plsc.addupdate(acc_v.at[c_first, pl.ds(j * _L, _L)],
                                       acc[j])
                    plsc.addupdate(cacc_v.at[c_first, :], chunkvec)

                @pl.when(c_first != c_last)
                def _chunk_mixed():
                    for k in range(_L):
                        g = gvec[k]
                        r = t * _L + k
                        for j in range(_CH):
                            plsc.addupdate(acc_v.at[g, pl.ds(j * _L, _L)],
                                           stage_v[r, pl.ds(j * _L, _L)])
                        plsc.addupdate(cacc_v.at[g, :], onevec)
                return carry
            lax.fori_loop(0, rows // _L, _chunk16, 0)

    for q in range(_NBUF - 1):
        @pl.when(q < nblk)
        def _prime(q=q):
            ci, ce = _copies(q, q)
            ci.start()
            ce.start()

    def _zero_row(i, carry):
        for j in range(_CH):
            acc_v[i, pl.ds(j * _L, _L)] = zvec
        cacc_v[i, :] = zvec
        return carry
    lax.fori_loop(0, _NUM_GRAPHS, _zero_row, 0)

    def _block(i, carry):
        for p in range(_NBUF):
            @pl.when(jnp.bitwise_and(i, _NBUF - 1) == p)
            def _run(p=p):
                ci, ce = _copies(i, p)
                ci.wait()
                ce.wait()

                @pl.when(i + _NBUF - 1 < nblk)
                def _prefetch():
                    ni, ne = _copies(i + _NBUF - 1,
                                     (p + _NBUF - 1) % _NBUF)
                    ni.start()
                    ne.start()

                _accumulate(p, _B)
        return carry

    lax.fori_loop(0, nblk, _block, 0)

    pltpu.sync_copy(acc_v, sums_hbm.at[cid, sid])
    pltpu.sync_copy(cacc_v, cnts_hbm.at[cid, sid])


_sc_partial = functools.partial(
    pl.kernel,
    out_type=(
        jax.ShapeDtypeStruct((_NC, _NS, _NUM_GRAPHS, _D), jnp.float32),
        jax.ShapeDtypeStruct((_NC, _NS, _NUM_GRAPHS, _L), jnp.float32),
    ),
    mesh=plsc.VectorSubcoreMesh(core_axis_name="c", subcore_axis_name="s"),
    scratch_types=(
        [pltpu.VMEM((_B, _D), jnp.float32)] * 4
        + [pltpu.VMEM((_B,), jnp.int32)] * 4
        + [
            pltpu.VMEM((_NUM_GRAPHS, _D), jnp.float32),
            pltpu.VMEM((_NUM_GRAPHS, _L), jnp.float32),
        ]
        + [pltpu.SemaphoreType.DMA] * 4
    ),
)(_sc_body)


def _tc_body(emb_ref, b_ref, s_ref, c_ref):
    i = pl.program_id(0)
    b = b_ref[0]
    gids = lax.broadcasted_iota(jnp.int32, (_NUM_GRAPHS, 1), 0)
    onehot = (b == gids).astype(jnp.float32)
    part = lax.dot(onehot, emb_ref[...],
                   precision=lax.Precision.HIGHEST,
                   preferred_element_type=jnp.float32)
    cnt = jnp.sum(onehot, axis=1, keepdims=True)

    @pl.when(i == 0)
    def _init():
        s_ref[...] = jnp.zeros_like(s_ref)
        c_ref[...] = jnp.zeros_like(c_ref)

    s_ref[...] += part
    c_ref[...] += cnt


def _tc_partial(node_emb, batch3):
    return pl.pallas_call(
        _tc_body,
        grid=(_TSTEPS,),
        in_specs=[
            pl.BlockSpec((_TB, _D), lambda i: (_TOFF + i, 0)),
            pl.BlockSpec((1, 1, _TB), lambda i: (_TOFF + i, 0, 0)),
        ],
        out_specs=(
            pl.BlockSpec((_NUM_GRAPHS, _D), lambda i: (0, 0)),
            pl.BlockSpec((_NUM_GRAPHS, 1), lambda i: (0, 0)),
        ),
        out_shape=(
            jax.ShapeDtypeStruct((_NUM_GRAPHS, _D), jnp.float32),
            jax.ShapeDtypeStruct((_NUM_GRAPHS, 1), jnp.float32),
        ),
    )(node_emb, batch3)


def _combine_body(s_ref, c_ref, st_ref, ct_ref, o_ref):
    s = jnp.sum(s_ref[...], axis=0) + st_ref[...]
    c = jnp.sum(c_ref[...], axis=0)[:, 0:1] + ct_ref[...]
    o_ref[...] = s / jnp.maximum(c, 1.0)


def _combine(sums_p, cnts_p, sums_t, cnts_t):
    return pl.pallas_call(
        _combine_body,
        out_shape=jax.ShapeDtypeStruct((_NUM_GRAPHS, _D), jnp.float32),
    )(sums_p.reshape(_NW, _NUM_GRAPHS, _D),
      cnts_p.reshape(_NW, _NUM_GRAPHS, _L),
      sums_t, cnts_t)


def kernel(node_emb, batch):
    sums_p, cnts_p = _sc_partial(node_emb, batch)
    sums_t, cnts_t = _tc_partial(node_emb, batch.reshape(-1, 1, _TB))
    return _combine(sums_p, cnts_p, sums_t, cnts_t)


# speedup vs baseline: 1.4316x; 1.4316x over previous
import functools

import jax
import jax.numpy as jnp
from jax import lax
from jax.experimental import pallas as pl
from jax.experimental.pallas import tpu as pltpu
from jax.experimental.pallas import tpu_sc as plsc

_NUM_GRAPHS = 64
_N_NODES = 50000
_D = 256
_L = 16
_CH = _D // _L

_B = 80
_NC = 2
_NS = 16
_NW = _NC * _NS

_SC_ROWS = 25600
_NBLOCKS = _SC_ROWS // _B
_TC_ROWS = _N_NODES - _SC_ROWS
_TB = 2440
_TSTEPS = _TC_ROWS // _TB


_NBUF = 4


def _sc_body(emb_hbm, batch_hbm,
             sums_hbm, cnts_hbm, stage0_v, stage1_v, stage2_v, stage3_v,
             idx0_v, idx1_v, idx2_v, idx3_v,
             acc_v, cacc_v, sem0, sem1, sem2, sem3):
    cid = lax.axis_index("c")
    sid = lax.axis_index("s")
    wid = sid * _NC + cid

    stages = (stage0_v, stage1_v, stage2_v, stage3_v)
    idxs = (idx0_v, idx1_v, idx2_v, idx3_v)
    sems = (sem0, sem1, sem2, sem3)

    nblk = (_NBLOCKS - 1 - wid) // _NW + 1

    onevec = jnp.ones((_L,), jnp.float32)

    def _copies(i, p):
        base = _TC_ROWS + (wid + i * _NW) * _B
        ci = pltpu.make_async_copy(
            batch_hbm.at[pl.ds(base, _B)], idxs[p], sems[p])
        ce = pltpu.make_async_copy(
            emb_hbm.at[pl.ds(base, _B)], stages[p], sems[p])
        return ci, ce

    zvec = jnp.zeros((_L,), jnp.float32)

    def _accumulate(p, rows):
        stage_v = stages[p]
        idx_v = idxs[p]
        blockvec = jnp.full((_L,), float(rows), jnp.float32)
        chunkvec = jnp.full((_L,), float(_L), jnp.float32)

        g_first = idx_v[pl.ds(0, _L)][0]
        g_last = idx_v[pl.ds(rows - _L, _L)][_L - 1]

        @pl.when(g_first == g_last)
        def _uniform():
            def _row(r, acc):
                return tuple(acc[j] + stage_v[r, pl.ds(j * _L, _L)]
                             for j in range(_CH))
            acc = lax.fori_loop(0, rows, _row, (zvec,) * _CH)
            for j in range(_CH):
                plsc.addupdate(acc_v.at[g_first, pl.ds(j * _L, _L)], acc[j])
            plsc.addupdate(cacc_v.at[g_first, :], blockvec)

        @pl.when(g_first != g_last)
        def _mixed():
            def _chunk16(t, carry):
                gvec = idx_v[pl.ds(t * _L, _L)]
                c_first = gvec[0]
                c_last = gvec[_L - 1]

                @pl.when(c_first == c_last)
                def _chunk_uniform():
                    def _row(r, acc):
                        return tuple(acc[j] + stage_v[r, pl.ds(j * _L, _L)]
                                     for j in range(_CH))
                    acc = lax.fori_loop(t * _L, (t + 1) * _L, _row,
                                        (zvec,) * _CH)
                    for j in range(_CH):
                        plsc.addupdate(acc_v.at[c_first, pl.ds(j * _L, _L)],
                                       acc[j])
                    plsc.addupdate(cacc_v.at[c_first, :], chunkvec)

                @pl.when(c_first != c_last)
                def _chunk_mixed():
                    for k in range(_L):
                        g = gvec[k]
                        r = t * _L + k
                        for j in range(_CH):
                            plsc.addupdate(acc_v.at[g, pl.ds(j * _L, _L)],
                                           stage_v[r, pl.ds(j * _L, _L)])
                        plsc.addupdate(cacc_v.at[g, :], onevec)
                return carry
            lax.fori_loop(0, rows // _L, _chunk16, 0)

    for q in range(_NBUF - 1):
        @pl.when(q < nblk)
        def _prime(q=q):
            ci, ce = _copies(q, q)
            ci.start()
            ce.start()

    def _zero_row(i, carry):
        for j in range(_CH):
            acc_v[i, pl.ds(j * _L, _L)] = zvec
        cacc_v[i, :] = zvec
        return carry
    lax.fori_loop(0, _NUM_GRAPHS, _zero_row, 0)

    def _block(i, carry):
        for p in range(_NBUF):
            @pl.when(jnp.bitwise_and(i, _NBUF - 1) == p)
            def _run(p=p):
                ci, ce = _copies(i, p)
                ci.wait()
                ce.wait()

                @pl.when(i + _NBUF - 1 < nblk)
                def _prefetch():
                    ni, ne = _copies(i + _NBUF - 1,
                                     (p + _NBUF - 1) % _NBUF)
                    ni.start()
                    ne.start()

                _accumulate(p, _B)
        return carry

    lax.fori_loop(0, nblk, _block, 0)

    pltpu.sync_copy(acc_v, sums_hbm.at[cid, sid])
    pltpu.sync_copy(cacc_v, cnts_hbm.at[cid, sid])


_sc_partial = functools.partial(
    pl.kernel,
    out_type=(
        jax.ShapeDtypeStruct((_NC, _NS, _NUM_GRAPHS, _D), jnp.float32),
        jax.ShapeDtypeStruct((_NC, _NS, _NUM_GRAPHS, _L), jnp.float32),
    ),
    mesh=plsc.VectorSubcoreMesh(core_axis_name="c", subcore_axis_name="s"),
    scratch_types=(
        [pltpu.VMEM((_B, _D), jnp.float32)] * 4
        + [pltpu.VMEM((_B,), jnp.int32)] * 4
        + [
            pltpu.VMEM((_NUM_GRAPHS, _D), jnp.float32),
            pltpu.VMEM((_NUM_GRAPHS, _L), jnp.float32),
        ]
        + [pltpu.SemaphoreType.DMA] * 4
    ),
)(_sc_body)


def _tc_body(emb_ref, b_ref, s_ref, c_ref):
    i = pl.program_id(0)
    b = b_ref[0]
    gids = lax.broadcasted_iota(jnp.int32, (_NUM_GRAPHS, 1), 0)
    onehot = (b == gids).astype(jnp.float32)
    part = lax.dot(onehot, emb_ref[...],
                   precision=lax.Precision.HIGHEST,
                   preferred_element_type=jnp.float32)
    cnt = jnp.sum(onehot, axis=1, keepdims=True)

    @pl.when(i == 0)
    def _init():
        s_ref[...] = jnp.zeros_like(s_ref)
        c_ref[...] = jnp.zeros_like(c_ref)

    s_ref[...] += part
    c_ref[...] += cnt


def _tc_partial(node_emb, batch3):
    return pl.pallas_call(
        _tc_body,
        grid=(_TSTEPS,),
        in_specs=[
            pl.BlockSpec((_TB, _D), lambda i: (i, 0)),
            pl.BlockSpec((1, 1, _TB), lambda i: (i, 0, 0)),
        ],
        out_specs=(
            pl.BlockSpec((_NUM_GRAPHS, _D), lambda i: (0, 0)),
            pl.BlockSpec((_NUM_GRAPHS, 1), lambda i: (0, 0)),
        ),
        out_shape=(
            jax.ShapeDtypeStruct((_NUM_GRAPHS, _D), jnp.float32),
            jax.ShapeDtypeStruct((_NUM_GRAPHS, 1), jnp.float32),
        ),
    )(node_emb, batch3)


def _combine_body(s_ref, c_ref, st_ref, ct_ref, o_ref):
    s = jnp.sum(s_ref[...], axis=0) + st_ref[...]
    c = jnp.sum(c_ref[...], axis=0)[:, 0:1] + ct_ref[...]
    o_ref[...] = s / jnp.maximum(c, 1.0)


def _combine(sums_p, cnts_p, sums_t, cnts_t):
    return pl.pallas_call(
        _combine_body,
        out_shape=jax.ShapeDtypeStruct((_NUM_GRAPHS, _D), jnp.float32),
    )(sums_p.reshape(_NW, _NUM_GRAPHS, _D),
      cnts_p.reshape(_NW, _NUM_GRAPHS, _L),
      sums_t, cnts_t)


def kernel(node_emb, batch):
    sums_p, cnts_p = _sc_partial(node_emb, batch)
    sums_t, cnts_t = _tc_partial(
        node_emb, batch[:_TC_ROWS].reshape(_TSTEPS, 1, _TB))
    return _combine(sums_p, cnts_p, sums_t, cnts_t)


# speedup vs baseline: 1.4826x; 1.0356x over previous
import functools

import jax
import jax.numpy as jnp
from jax import lax
from jax.experimental import pallas as pl
from jax.experimental.pallas import tpu as pltpu
from jax.experimental.pallas import tpu_sc as plsc

_NUM_GRAPHS = 64
_N_NODES = 50000
_D = 256
_L = 16
_CH = _D // _L

_B = 80
_NC = 2
_NS = 16
_NW = _NC * _NS

_SC_ROWS = 20480
_NBLOCKS = _SC_ROWS // _B
_TC_ROWS = _N_NODES - _SC_ROWS
_TB = 2952
_TSTEPS = _TC_ROWS // _TB


_NBUF = 4


def _sc_body(emb_hbm, batch_hbm,
             sums_hbm, cnts_hbm, stage0_v, stage1_v, stage2_v, stage3_v,
             idx0_v, idx1_v, idx2_v, idx3_v,
             acc_v, cacc_v, sem0, sem1, sem2, sem3):
    cid = lax.axis_index("c")
    sid = lax.axis_index("s")
    wid = sid * _NC + cid

    stages = (stage0_v, stage1_v, stage2_v, stage3_v)
    idxs = (idx0_v, idx1_v, idx2_v, idx3_v)
    sems = (sem0, sem1, sem2, sem3)

    nblk = (_NBLOCKS - 1 - wid) // _NW + 1

    onevec = jnp.ones((_L,), jnp.float32)

    def _copies(i, p):
        base = _TC_ROWS + (wid + i * _NW) * _B
        ci = pltpu.make_async_copy(
            batch_hbm.at[pl.ds(base, _B)], idxs[p], sems[p])
        ce = pltpu.make_async_copy(
            emb_hbm.at[pl.ds(base, _B)], stages[p], sems[p])
        return ci, ce

    zvec = jnp.zeros((_L,), jnp.float32)

    def _accumulate(p, rows):
        stage_v = stages[p]
        idx_v = idxs[p]
        blockvec = jnp.full((_L,), float(rows), jnp.float32)
        chunkvec = jnp.full((_L,), float(_L), jnp.float32)

        g_first = idx_v[pl.ds(0, _L)][0]
        g_last = idx_v[pl.ds(rows - _L, _L)][_L - 1]

        @pl.when(g_first == g_last)
        def _uniform():
            def _row(r, acc):
                return tuple(acc[j] + stage_v[r, pl.ds(j * _L, _L)]
                             for j in range(_CH))
            acc = lax.fori_loop(0, rows, _row, (zvec,) * _CH)
            for j in range(_CH):
                plsc.addupdate(acc_v.at[g_first, pl.ds(j * _L, _L)], acc[j])
            plsc.addupdate(cacc_v.at[g_first, :], blockvec)

        @pl.when(g_first != g_last)
        def _mixed():
            def _chunk16(t, carry):
                gvec = idx_v[pl.ds(t * _L, _L)]
                c_first = gvec[0]
                c_last = gvec[_L - 1]

                @pl.when(c_first == c_last)
                def _chunk_uniform():
                    def _row(r, acc):
                        return tuple(acc[j] + stage_v[r, pl.ds(j * _L, _L)]
                                     for j in range(_CH))
                    acc = lax.fori_loop(t * _L, (t + 1) * _L, _row,
                                        (zvec,) * _CH)
                    for j in range(_CH):
                        plsc.addupdate(acc_v.at[c_first, pl.ds(j * _L, _L)],
                                       acc[j])
                    plsc.addupdate(cacc_v.at[c_first, :], chunkvec)

                @pl.when(c_first != c_last)
                def _chunk_mixed():
                    for k in range(_L):
                        g = gvec[k]
                        r = t * _L + k
                        for j in range(_CH):
                            plsc.addupdate(acc_v.at[g, pl.ds(j * _L, _L)],
                                           stage_v[r, pl.ds(j * _L, _L)])
                        plsc.addupdate(cacc_v.at[g, :], onevec)
                return carry
            lax.fori_loop(0, rows // _L, _chunk16, 0)

    for q in range(_NBUF - 1):
        @pl.when(q < nblk)
        def _prime(q=q):
            ci, ce = _copies(q, q)
            ci.start()
            ce.start()

    def _zero_row(i, carry):
        for j in range(_CH):
            acc_v[i, pl.ds(j * _L, _L)] = zvec
        cacc_v[i, :] = zvec
        return carry
    lax.fori_loop(0, _NUM_GRAPHS, _zero_row, 0)

    def _block(i, carry):
        for p in range(_NBUF):
            @pl.when(jnp.bitwise_and(i, _NBUF - 1) == p)
            def _run(p=p):
                ci, ce = _copies(i, p)
                ci.wait()
                ce.wait()

                @pl.when(i + _NBUF - 1 < nblk)
                def _prefetch():
                    ni, ne = _copies(i + _NBUF - 1,
                                     (p + _NBUF - 1) % _NBUF)
                    ni.start()
                    ne.start()

                _accumulate(p, _B)
        return carry

    lax.fori_loop(0, nblk, _block, 0)

    pltpu.sync_copy(acc_v, sums_hbm.at[cid, sid])
    pltpu.sync_copy(cacc_v, cnts_hbm.at[cid, sid])


_sc_partial = functools.partial(
    pl.kernel,
    out_type=(
        jax.ShapeDtypeStruct((_NC, _NS, _NUM_GRAPHS, _D), jnp.float32),
        jax.ShapeDtypeStruct((_NC, _NS, _NUM_GRAPHS, _L), jnp.float32),
    ),
    mesh=plsc.VectorSubcoreMesh(core_axis_name="c", subcore_axis_name="s"),
    scratch_types=(
        [pltpu.VMEM((_B, _D), jnp.float32)] * 4
        + [pltpu.VMEM((_B,), jnp.int32)] * 4
        + [
            pltpu.VMEM((_NUM_GRAPHS, _D), jnp.float32),
            pltpu.VMEM((_NUM_GRAPHS, _L), jnp.float32),
        ]
        + [pltpu.SemaphoreType.DMA] * 4
    ),
)(_sc_body)


def _tc_body(emb_ref, b_ref, s_ref, c_ref):
    i = pl.program_id(0)
    b = b_ref[0]
    gids = lax.broadcasted_iota(jnp.int32, (_NUM_GRAPHS, 1), 0)
    onehot = (b == gids).astype(jnp.float32)
    part = lax.dot(onehot, emb_ref[...],
                   precision=lax.Precision.HIGHEST,
                   preferred_element_type=jnp.float32)
    cnt = jnp.sum(onehot, axis=1, keepdims=True)

    @pl.when(i == 0)
    def _init():
        s_ref[...] = jnp.zeros_like(s_ref)
        c_ref[...] = jnp.zeros_like(c_ref)

    s_ref[...] += part
    c_ref[...] += cnt


def _tc_partial(node_emb, batch3):
    return pl.pallas_call(
        _tc_body,
        grid=(_TSTEPS,),
        in_specs=[
            pl.BlockSpec((_TB, _D), lambda i: (i, 0)),
            pl.BlockSpec((1, 1, _TB), lambda i: (i, 0, 0)),
        ],
        out_specs=(
            pl.BlockSpec((_NUM_GRAPHS, _D), lambda i: (0, 0)),
            pl.BlockSpec((_NUM_GRAPHS, 1), lambda i: (0, 0)),
        ),
        out_shape=(
            jax.ShapeDtypeStruct((_NUM_GRAPHS, _D), jnp.float32),
            jax.ShapeDtypeStruct((_NUM_GRAPHS, 1), jnp.float32),
        ),
    )(node_emb, batch3)


def _combine_body(s_ref, c_ref, st_ref, ct_ref, o_ref):
    s = jnp.sum(s_ref[...], axis=0) + st_ref[...]
    c = jnp.sum(c_ref[...], axis=0)[:, 0:1] + ct_ref[...]
    o_ref[...] = s / jnp.maximum(c, 1.0)


def _combine(sums_p, cnts_p, sums_t, cnts_t):
    return pl.pallas_call(
        _combine_body,
        out_shape=jax.ShapeDtypeStruct((_NUM_GRAPHS, _D), jnp.float32),
    )(sums_p.reshape(_NW, _NUM_GRAPHS, _D),
      cnts_p.reshape(_NW, _NUM_GRAPHS, _L),
      sums_t, cnts_t)


def kernel(node_emb, batch):
    sums_p, cnts_p = _sc_partial(node_emb, batch)
    sums_t, cnts_t = _tc_partial(
        node_emb, batch[:_TC_ROWS].reshape(_TSTEPS, 1, _TB))
    return _combine(sums_p, cnts_p, sums_t, cnts_t)
